# baseline (speedup 1.0000x reference)
import jax
import jax.numpy as jnp
from jax import lax
from jax.experimental import pallas as pl
from jax.experimental.pallas import tpu as pltpu
from jax.experimental.pallas import tpu_sc as plsc

NUM_SCIENTISTS = 100000
NUM_PAPERS = 1000000
D = 64
GLOBAL_MEAN = 3.5
IMPLICIT_WEIGHT = 0.5
B = 4096
L = 50

NC, NS = 2, 16
NW = NC * NS
E = B // NW
C = 16
NCH = E // C
DV = D // 16
HC = C // 2
HROWS = HC * L
NG = NCH * 2
UNIT = 80
NUNIT = HROWS // UNIT


def _ext(v, i):
    return lax.squeeze(lax.slice(v, (i,), (i + 1,)), dimensions=(0,))


def _vgather(x, idx):
    dn = lax.GatherDimensionNumbers(
        offset_dims=(), collapsed_slice_dims=(0,), start_index_map=(0,))
    return lax.gather(x, idx[:, None], dn, (1,),
                      mode=lax.GatherScatterMode.PROMISE_IN_BOUNDS)


def _splat(x, i):
    return _vgather(x, jnp.full((16,), i, dtype=jnp.int32))


def _lanesum(t, lane):
    for k in (8, 4, 2, 1):
        t = t + _vgather(t, lane ^ jnp.int32(k))
    return t


def _sc_body(sids_hbm, pids_hbm, imp_hbm, lens_hbm, starts_hbm, nu_hbm,
             p_hbm, q_hbm, bs_hbm, bp_hbm, out_hbm,
             sids_v, pids_v, lens_v, impf_v, starts_v, nu_v,
             rows0_v, rows1_v, p0_v, p1_v, q0_v, q1_v,
             bs0_v, bs1_v, bp0_v, bp1_v, out_v, sem0, sem1):
    cid = lax.axis_index("c")
    scid = lax.axis_index("s")
    wid = scid * NC + cid

    pltpu.sync_copy(sids_hbm.at[wid], sids_v)
    pltpu.sync_copy(pids_hbm.at[wid], pids_v)
    pltpu.sync_copy(lens_hbm.at[wid], lens_v)
    pltpu.sync_copy(imp_hbm.at[wid], impf_v)
    pltpu.sync_copy(starts_hbm.at[wid], starts_v)
    pltpu.sync_copy(nu_hbm.at[wid], nu_v)


    lane = lax.iota(jnp.int32, 16)
    rbufs = (rows0_v, rows1_v)
    sems = (sem0, sem1)
    pqbufs = ((p0_v, q0_v, bs0_v, bp0_v), (p1_v, q1_v, bs1_v, bp1_v))

    def pq_pairs(g, pqk):
        ch = g // 2
        pv, qv, bsv, bpv = pqbufs[pqk]
        return [(p_hbm.at[sids_v.at[ch]], pv),
                (q_hbm.at[pids_v.at[ch]], qv),
                (bs_hbm.at[sids_v.at[ch]], bsv),
                (bp_hbm.at[pids_v.at[ch]], bpv)]

    def unit_pairs(g, rk, u):
        ch = g // 2
        half = g % 2
        src = q_hbm.at[impf_v.at[ch, pl.ds(half * HROWS + u * UNIT, UNIT)]]
        return src, rbufs[rk].at[pl.ds(u * UNIT, UNIT)]

    def fire(g, rk, pqk, even, halfs):
        ch = g // 2
        n = _ext(nu_v[ch, :], halfs)
        for u in range(NUNIT):
            @pl.when(jnp.int32(u) < n)
            def _f():
                s, d = unit_pairs(g, rk, u)
                pltpu.async_copy(s, d, sems[rk])
        if even:
            for s, d in pq_pairs(g, pqk):
                pltpu.async_copy(s, d, sems[rk])

    def drain(g, rk, pqk, even, halfs):
        ch = g // 2
        n = _ext(nu_v[ch, :], halfs)
        for u in range(NUNIT):
            @pl.when(jnp.int32(u) < n)
            def _d():
                s, d = unit_pairs(g, rk, u)
                pltpu.make_async_copy(s, d, sems[rk]).wait()
        if even:
            for s, d in pq_pairs(g, pqk):
                pltpu.make_async_copy(s, d, sems[rk]).wait()

    def compute(g, rk, pqk, half):
        ch = g // 2
        rows_v = rbufs[rk]
        p_v, q_v, bs_v, bp_v = pqbufs[pqk]
        lens = lens_v[ch, :]
        lens_f = lens.astype(jnp.float32)
        h = 0.5 * lens_f
        yb = jnp.int32(0x5F3759DF) - (lax.bitcast_convert_type(
            lens_f, jnp.int32) >> 1)
        y = lax.bitcast_convert_type(yb, jnp.float32)
        for _ in range(3):
            y = y * (1.5 - h * y * y)
        sqrt_n = lens_f * y
        alpha = IMPLICIT_WEIGHT / (sqrt_n + 1e-9)

        starts_vec = starts_v[ch, :]
        zero = jnp.zeros((16,), jnp.float32)
        one = jnp.int32(1)
        zeroi = jnp.int32(0)
        out_vec = zero
        for i in range(half * HC, half * HC + HC):
            a_i = _splat(alpha, jnp.int32(i))
            st = _ext(starts_vec, i)
            ln = _ext(lens, i)

            def lstep(l, accs, st=st):
                r = st + l
                return tuple(
                    accs[d] + rows_v[r, pl.ds(d * 16, 16)]
                    for d in range(DV))

            acc = lax.fori_loop(0, ln, lstep, (zero, zero, zero, zero))
            t = zero
            for d in range(DV):
                u = p_v[i, pl.ds(d * 16, 16)] + a_i * acc[d]
                t = t + q_v[i, pl.ds(d * 16, 16)] * u
            tot = _lanesum(t, lane)
            eq = jnp.minimum(jnp.maximum(one - jnp.abs(lane - i), zeroi),
                             one).astype(jnp.float32)
            out_vec = out_vec + eq * tot
        if half == 0:
            out_v[ch, :] = GLOBAL_MEAN + bs_v[:] + bp_v[:] + out_vec
        else:
            out_v[ch, :] = out_v[ch, :] + out_vec

    fire(jnp.int32(0), 0, 0, True, 0)

    def body(s, carry):
        for j in range(4):
            g = 4 * s + j
            rk = j % 2
            pqk = j // 2
            half = j % 2
            gn = jnp.minimum(g + 1, NG - 1)
            rkn = (j + 1) % 2
            pqkn = ((j + 1) // 2) % 2
            evenn = ((j + 1) % 2 == 0)
            drain(g, rk, pqk, half == 0, j % 2)
            fire(gn, rkn, pqkn, evenn, (j + 1) % 2)
            compute(g, rk, pqk, half)
        return carry

    lax.fori_loop(0, NG // 4, body, jnp.int32(0))
    drain(jnp.int32(NG - 1), 0, 0, True, 0)
    pltpu.sync_copy(out_v, out_hbm.at[wid])


def kernel(SIDs, PIDs, implicit_PIDs, implicit_lengths, P, Q,
           scientist_bias, paper_bias):
    mesh = plsc.VectorSubcoreMesh(core_axis_name="c", subcore_axis_name="s",
                                  num_cores=NC, num_subcores=NS)
    run = pl.kernel(
        _sc_body,
        out_type=jax.ShapeDtypeStruct((NW, NCH, C), jnp.float32),
        mesh=mesh,
        compiler_params=pltpu.CompilerParams(use_tc_tiling_on_sc=False),
        scratch_types=[
            pltpu.VMEM((NCH, C), jnp.int32),
            pltpu.VMEM((NCH, C), jnp.int32),
            pltpu.VMEM((NCH, C), jnp.int32),
            pltpu.VMEM((NCH, C * L), jnp.int32),
            pltpu.VMEM((NCH, C), jnp.int32),
            pltpu.VMEM((NCH, 16), jnp.int32),
            pltpu.VMEM((HROWS, 128), jnp.float32),
            pltpu.VMEM((HROWS, 128), jnp.float32),
            pltpu.VMEM((C, 128), jnp.float32),
            pltpu.VMEM((C, 128), jnp.float32),
            pltpu.VMEM((C, 128), jnp.float32),
            pltpu.VMEM((C, 128), jnp.float32),
            pltpu.VMEM((C,), jnp.float32),
            pltpu.VMEM((C,), jnp.float32),
            pltpu.VMEM((C,), jnp.float32),
            pltpu.VMEM((C,), jnp.float32),
            pltpu.VMEM((NCH, C), jnp.float32),
            pltpu.SemaphoreType.DMA,
            pltpu.SemaphoreType.DMA,
        ],
    )
    Qp = jnp.pad(Q, ((0, 0), (0, 128 - D)))
    Pp = jnp.pad(P, ((0, 0), (0, 128 - D)))
    lens_i = implicit_lengths.astype(jnp.int32)
    imp = implicit_PIDs.reshape(B // HC, HC, L).astype(jnp.int32)
    valid = (jnp.arange(L, dtype=jnp.int32)[None, None, :]
             < lens_i.reshape(B // HC, HC, 1))
    flat_valid = valid.reshape(B // HC, HC * L)
    order = jnp.argsort(~flat_valid, axis=1, stable=True)
    cmp_ids = jnp.take_along_axis(imp.reshape(B // HC, HC * L), order, axis=1)
    lens_h = lens_i.reshape(B // HC, HC)
    csum = jnp.cumsum(lens_h, axis=1)
    starts = (csum - lens_h).astype(jnp.int32)
    total = csum[:, -1]
    nunits = ((total + UNIT - 1) // UNIT).astype(jnp.int32)
    out = run(
        SIDs.reshape(NW, NCH, C).astype(jnp.int32),
        PIDs.reshape(NW, NCH, C).astype(jnp.int32),
        cmp_ids.reshape(NW, NCH, C * L),
        lens_i.reshape(NW, NCH, C),
        starts.reshape(NW, NCH, C),
        jnp.pad(nunits.reshape(NW, NCH, 2), ((0, 0), (0, 0), (0, 14))),
        Pp,
        Qp,
        scientist_bias.reshape(NUM_SCIENTISTS),
        paper_bias.reshape(NUM_PAPERS),
    )
    return out.reshape(B)


# speedup vs baseline: 1.0327x; 1.0327x over previous
import jax
import jax.numpy as jnp
from jax import lax
from jax.experimental import pallas as pl
from jax.experimental.pallas import tpu as pltpu
from jax.experimental.pallas import tpu_sc as plsc

NUM_SCIENTISTS = 100000
NUM_PAPERS = 1000000
D = 64
GLOBAL_MEAN = 3.5
IMPLICIT_WEIGHT = 0.5
B = 4096
L = 50

NC, NS = 2, 16
NW = NC * NS
E = B // NW
C = 16
NCH = E // C
DV = D // 16
HC = C // 2
HROWS = HC * L
NG = NCH * 2


def _vgather(x, idx):
    dn = lax.GatherDimensionNumbers(
        offset_dims=(), collapsed_slice_dims=(0,), start_index_map=(0,))
    return lax.gather(x, idx[:, None], dn, (1,),
                      mode=lax.GatherScatterMode.PROMISE_IN_BOUNDS)


def _splat(x, i):
    return _vgather(x, jnp.full((16,), i, dtype=jnp.int32))


def _lanesum(t, lane):
    for k in (8, 4, 2, 1):
        t = t + _vgather(t, lane ^ jnp.int32(k))
    return t


def _sc_body(sids_hbm, pids_hbm, imp_hbm, lens_hbm, p_hbm, q_hbm,
             bs_hbm, bp_hbm, out_hbm,
             sids_v, pids_v, lens_v, impf_v,
             rows0_v, rows1_v, p0_v, p1_v, q0_v, q1_v,
             bs0_v, bs1_v, bp0_v, bp1_v, out_v, sem0, sem1):
    cid = lax.axis_index("c")
    scid = lax.axis_index("s")
    wid = scid * NC + cid

    pltpu.sync_copy(sids_hbm.at[wid], sids_v)
    pltpu.sync_copy(pids_hbm.at[wid], pids_v)
    pltpu.sync_copy(lens_hbm.at[wid], lens_v)
    pltpu.sync_copy(imp_hbm.at[wid], impf_v)

    lane = lax.iota(jnp.int32, 16)
    rbufs = (rows0_v, rows1_v)
    sems = (sem0, sem1)
    pqbufs = ((p0_v, q0_v, bs0_v, bp0_v), (p1_v, q1_v, bs1_v, bp1_v))

    def pairs(g, rk, pqk, even):
        ch = g // 2
        half = g % 2
        prs = [(q_hbm.at[impf_v.at[ch, pl.ds((g % 2) * HROWS, HROWS)]],
                rbufs[rk])]
        if even:
            pv, qv, bsv, bpv = pqbufs[pqk]
            prs += [(p_hbm.at[sids_v.at[ch]], pv),
                    (q_hbm.at[pids_v.at[ch]], qv),
                    (bs_hbm.at[sids_v.at[ch]], bsv),
                    (bp_hbm.at[pids_v.at[ch]], bpv)]
        return prs, sems[rk]

    def fire(g, rk, pqk, even):
        prs, sem = pairs(g, rk, pqk, even)
        for s, d in prs:
            pltpu.async_copy(s, d, sem)

    def drain(g, rk, pqk, even):
        prs, sem = pairs(g, rk, pqk, even)
        for s, d in prs:
            pltpu.make_async_copy(s, d, sem).wait()

    def compute(g, rk, pqk, half):
        ch = g // 2
        rows_v = rbufs[rk]
        p_v, q_v, bs_v, bp_v = pqbufs[pqk]
        lens = lens_v[ch, :]
        lens_f = lens.astype(jnp.float32)
        h = 0.5 * lens_f
        yb = jnp.int32(0x5F3759DF) - (lax.bitcast_convert_type(
            lens_f, jnp.int32) >> 1)
        y = lax.bitcast_convert_type(yb, jnp.float32)
        for _ in range(3):
            y = y * (1.5 - h * y * y)
        sqrt_n = lens_f * y
        alpha = IMPLICIT_WEIGHT / (sqrt_n + 1e-9)

        def elem(i, out_vec):
            len_i = _splat(lens, i)
            a_i = _splat(alpha, i)
            zero = jnp.zeros((16,), jnp.float32)
            one = jnp.int32(1)
            zeroi = jnp.int32(0)
            loc = i - half * HC
            acc = [zero, zero, zero, zero]
            for l in range(L):
                mf = jnp.minimum(jnp.maximum(len_i - jnp.int32(l), zeroi),
                                 one).astype(jnp.float32)
                r = loc * L + l
                for d in range(DV):
                    acc[d] = acc[d] + mf * rows_v[r, pl.ds(d * 16, 16)]
            t = zero
            for d in range(DV):
                u = p_v[i, pl.ds(d * 16, 16)] + a_i * acc[d]
                t = t + q_v[i, pl.ds(d * 16, 16)] * u
            tot = _lanesum(t, lane)
            eq = jnp.minimum(jnp.maximum(one - jnp.abs(lane - i), zeroi),
                             one).astype(jnp.float32)
            return out_vec + eq * tot

        out_vec = lax.fori_loop(half * HC, half * HC + HC, elem,
                                jnp.zeros((16,), jnp.float32))
        if half == 0:
            out_v[ch, :] = GLOBAL_MEAN + bs_v[:] + bp_v[:] + out_vec
        else:
            out_v[ch, :] = out_v[ch, :] + out_vec

    fire(jnp.int32(0), 0, 0, True)

    def body(s, carry):
        for j in range(4):
            g = 4 * s + j
            rk = j % 2
            pqk = j // 2
            half = j % 2
            gn = jnp.minimum(g + 1, NG - 1)
            rkn = (j + 1) % 2
            pqkn = ((j + 1) // 2) % 2
            evenn = ((j + 1) % 2 == 0)
            drain(g, rk, pqk, half == 0)
            fire(gn, rkn, pqkn, evenn)
            compute(g, rk, pqk, half)
        return carry

    lax.fori_loop(0, NG // 4, body, jnp.int32(0))
    drain(jnp.int32(NG - 1), 0, 0, True)
    pltpu.sync_copy(out_v, out_hbm.at[wid])


def kernel(SIDs, PIDs, implicit_PIDs, implicit_lengths, P, Q,
           scientist_bias, paper_bias):
    mesh = plsc.VectorSubcoreMesh(core_axis_name="c", subcore_axis_name="s",
                                  num_cores=NC, num_subcores=NS)
    run = pl.kernel(
        _sc_body,
        out_type=jax.ShapeDtypeStruct((NW, NCH, C), jnp.float32),
        mesh=mesh,
        compiler_params=pltpu.CompilerParams(use_tc_tiling_on_sc=False),
        scratch_types=[
            pltpu.VMEM((NCH, C), jnp.int32),
            pltpu.VMEM((NCH, C), jnp.int32),
            pltpu.VMEM((NCH, C), jnp.int32),
            pltpu.VMEM((NCH, C * L), jnp.int32),
            pltpu.VMEM((HROWS, 128), jnp.float32),
            pltpu.VMEM((HROWS, 128), jnp.float32),
            pltpu.VMEM((C, 128), jnp.float32),
            pltpu.VMEM((C, 128), jnp.float32),
            pltpu.VMEM((C, 128), jnp.float32),
            pltpu.VMEM((C, 128), jnp.float32),
            pltpu.VMEM((C,), jnp.float32),
            pltpu.VMEM((C,), jnp.float32),
            pltpu.VMEM((C,), jnp.float32),
            pltpu.VMEM((C,), jnp.float32),
            pltpu.VMEM((NCH, C), jnp.float32),
            pltpu.SemaphoreType.DMA,
            pltpu.SemaphoreType.DMA,
        ],
    )
    Qp = jnp.pad(Q, ((0, 0), (0, 128 - D)))
    Pp = jnp.pad(P, ((0, 0), (0, 128 - D)))
    out = run(
        SIDs.reshape(NW, NCH, C).astype(jnp.int32),
        PIDs.reshape(NW, NCH, C).astype(jnp.int32),
        implicit_PIDs.reshape(NW, NCH, C * L).astype(jnp.int32),
        implicit_lengths.reshape(NW, NCH, C).astype(jnp.int32),
        Pp,
        Qp,
        scientist_bias.reshape(NUM_SCIENTISTS),
        paper_bias.reshape(NUM_PAPERS),
    )
    return out.reshape(B)
